# baseline (speedup 1.0000x reference)
import functools

import jax
import jax.numpy as jnp
from jax import lax
from jax.experimental import pallas as pl
from jax.experimental.pallas import tpu as pltpu
from jax.experimental.pallas import tpu_sc as plsc

EMBED = 64
SCALE = 8.0
VOCAB = 1000000

NC = 2
NS = 16
NW = NC * NS

ROWS = 4096
COLS = 200
TBLK = ROWS // NW

RB = 2048


NBANDS = VOCAB // 128
BREM = VOCAB - NBANDS * 128


@functools.partial(
    pl.kernel,
    mesh=plsc.VectorSubcoreMesh(core_axis_name="c", subcore_axis_name="s"),
    out_type=jax.ShapeDtypeStruct((VOCAB, 2 * EMBED), jnp.float32),
    scratch_types=[
        pltpu.VMEM((EMBED, 128), jnp.float32),
        pltpu.VMEM((EMBED, 128), jnp.float32),
        pltpu.VMEM((128, 129), jnp.float32),
        pltpu.VMEM((128, 129), jnp.float32),
        pltpu.SemaphoreType.DMA,
        pltpu.SemaphoreType.DMA,
        pltpu.SemaphoreType.DMA,
    ],
    compiler_params=pltpu.CompilerParams(needs_layout_passes=False),
)
def _pre_transpose(tt_hbm, rem_hbm, out_hbm, v0, v1, o0, o1, s0, s1, osem):
    wid = lax.axis_index("s") * NC + lax.axis_index("c")
    n = NBANDS // NW + jnp.where(wid < NBANDS % NW, 1, 0)
    base = wid * (NBANDS // NW) + jnp.minimum(wid, NBANDS % NW)

    iota = lax.iota(jnp.int32, 16)
    rowvs = [iota + g * 16 for g in range(8)]

    def fetch(r, vbuf, sem):
        pltpu.async_copy(tt_hbm.at[:, pl.ds(r * 128, 128)], vbuf, sem)

    def wait(vbuf, sem):
        pltpu.make_async_copy(tt_hbm.at[:, pl.ds(0, 128)], vbuf, sem).wait()

    def transform(vbuf, obuf):
        def c_body(c, _):
            colv = jnp.full((16,), 0, jnp.int32) + c
            for g in range(8):
                v = vbuf[c, pl.ds(g * 16, 16)]
                plsc.store_scatter(obuf, [rowvs[g], colv], v)
            return 0
        lax.fori_loop(0, EMBED, c_body, 0, unroll=4)

    def put(r, obuf):
        pltpu.async_copy(obuf.at[:, pl.ds(0, 128)],
                         out_hbm.at[pl.ds(r * 128, 128)], osem)

    def drain_put(obuf):
        pltpu.make_async_copy(out_hbm.at[pl.ds(0, 128)],
                              obuf.at[:, pl.ds(0, 128)], osem).wait()

    fetch(base, v0, s0)

    def pair_body(t, _):
        r0 = base + 2 * t

        @pl.when(2 * t < n)
        def _():
            @pl.when(2 * t + 1 < n)
            def _():
                fetch(r0 + 1, v1, s1)

            wait(v0, s0)

            @pl.when(t > 0)
            def _():
                drain_put(o0)

            transform(v0, o0)
            put(r0, o0)

            @pl.when(2 * t + 2 < n)
            def _():
                fetch(r0 + 2, v0, s0)

            @pl.when(2 * t + 1 < n)
            def _():
                wait(v1, s1)

                @pl.when(t > 0)
                def _():
                    drain_put(o1)

                transform(v1, o1)
                put(r0 + 1, o1)

        return 0

    lax.fori_loop(0, (NBANDS // NW + 2) // 2, pair_body, 0)
    drain_put(o0)

    @pl.when(n > 1)
    def _():
        drain_put(o1)

    @pl.when(wid == NW - 1)
    def _():
        pltpu.sync_copy(rem_hbm, v0)
        pltpu.sync_copy(v0, out_hbm.at[pl.ds(NBANDS * 128, BREM)])


def _retile(table):
    rem = jnp.pad(table[NBANDS * 128:], ((0, 0), (0, EMBED)))
    return _pre_transpose(table.T, rem)


@functools.partial(
    pl.kernel,
    mesh=plsc.VectorSubcoreMesh(core_axis_name="c", subcore_axis_name="s"),
    out_type=jax.ShapeDtypeStruct((COLS, 8, NW, 8, TBLK), jnp.float32),
    scratch_types=[
        pltpu.VMEM((COLS, TBLK), jnp.int32),
        pltpu.VMEM((TBLK, 2 * EMBED), jnp.float32),
        pltpu.VMEM((TBLK, 2 * EMBED), jnp.float32),
        pltpu.VMEM((8, 8, TBLK + 1), jnp.float32),
        pltpu.VMEM((8, 8, TBLK + 1), jnp.float32),
        pltpu.SemaphoreType.DMA,
        pltpu.SemaphoreType.DMA,
        pltpu.SemaphoreType.DMA,
    ],
    compiler_params=pltpu.CompilerParams(
        use_tc_tiling_on_sc=False, needs_layout_passes=False),
)
def _embed_gather(idxt_hbm, table_hbm, out_hbm, idxt_v, rows0_v, rows1_v,
                  t0_v, t1_v, sem0, sem1, osem):
    wid = lax.axis_index("s") * NC + lax.axis_index("c")

    pltpu.sync_copy(idxt_hbm.at[:, pl.ds(wid * TBLK, TBLK)], idxt_v)

    def issue(j, buf, sem):
        pltpu.async_copy(table_hbm.at[idxt_v.at[j]], buf, sem)

    def drain(buf, sem):
        pltpu.make_async_copy(table_hbm.at[idxt_v.at[0]], buf, sem).wait()

    iota = lax.iota(jnp.int32, 16)
    icl = jnp.bitwise_and(iota, 7)
    ic8_base = jnp.right_shift(iota, 3)
    igrp_c8 = [ic8_base + 2 * k for k in range(EMBED // 16)]
    igrp_cl = [icl for _ in range(EMBED // 16)]

    def transform(rows_v, t_v):
        def row_body(tt, _):
            tv = jnp.full((16,), 0, jnp.int32) + tt
            for k in range(EMBED // 16):
                v = rows_v[tt, pl.ds(k * 16, 16)]
                plsc.store_scatter(
                    t_v, [igrp_c8[k], igrp_cl[k], tv], v * SCALE)
            return 0
        lax.fori_loop(0, TBLK, row_body, 0, unroll=4)

    def store(j, t_v):
        for c8 in range(8):
            pltpu.async_copy(t_v.at[c8, :, pl.ds(0, TBLK)],
                             out_hbm.at[j, c8, wid], osem)

    def drain_store(t_v):
        for c8 in range(8):
            pltpu.make_async_copy(out_hbm.at[0, c8, wid],
                                  t_v.at[c8, :, pl.ds(0, TBLK)], osem).wait()

    issue(0, rows0_v, sem0)

    def pair_body(t, _):
        j0 = 2 * t
        issue(j0 + 1, rows1_v, sem1)
        drain(rows0_v, sem0)

        @pl.when(t > 0)
        def _():
            drain_store(t0_v)

        transform(rows0_v, t0_v)
        store(j0, t0_v)

        @pl.when(j0 + 2 < COLS)
        def _():
            issue(j0 + 2, rows0_v, sem0)

        drain(rows1_v, sem1)

        @pl.when(t > 0)
        def _():
            drain_store(t1_v)

        transform(rows1_v, t1_v)
        store(j0 + 1, t1_v)
        return 0

    lax.fori_loop(0, COLS // 2, pair_body, 0)
    drain_store(t0_v)
    drain_store(t1_v)


def kernel(tokens, table):
    tp = _retile(table)
    out5 = _embed_gather(tokens.astype(jnp.int32).T, tp)
    return out5.transpose(2, 4, 0, 1, 3).reshape(ROWS, COLS, EMBED)


# speedup vs baseline: 1.7782x; 1.7782x over previous
import functools

import jax
import jax.numpy as jnp
from jax import lax
from jax.experimental import pallas as pl
from jax.experimental.pallas import tpu as pltpu
from jax.experimental.pallas import tpu_sc as plsc

EMBED = 64
SCALE = 8.0
VOCAB = 1000000

NC = 2
NS = 16
NW = NC * NS

ROWS = 4096
COLS = 200
TBLK = ROWS // NW

RB = 2048


def _retile_body(tt_ref, out_ref):
    xt = tt_ref[...].T
    out_ref[...] = jnp.pad(xt, ((0, 0), (0, EMBED)))


_retile_call = pl.pallas_call(
    _retile_body,
    grid=(pl.cdiv(VOCAB, RB),),
    in_specs=[pl.BlockSpec((EMBED, RB), lambda i: (0, i))],
    out_specs=pl.BlockSpec((RB, 2 * EMBED), lambda i: (i, 0)),
    out_shape=jax.ShapeDtypeStruct((VOCAB, 2 * EMBED), jnp.float32),
)


def _retile(table):
    return _retile_call(table.T)


@functools.partial(
    pl.kernel,
    mesh=plsc.VectorSubcoreMesh(core_axis_name="c", subcore_axis_name="s"),
    out_type=jax.ShapeDtypeStruct((COLS, 8, NW, 8, TBLK), jnp.float32),
    scratch_types=[
        pltpu.VMEM((COLS, TBLK), jnp.int32),
        pltpu.VMEM((TBLK, 2 * EMBED), jnp.float32),
        pltpu.VMEM((TBLK, 2 * EMBED), jnp.float32),
        pltpu.VMEM((8, 8, TBLK + 1), jnp.float32),
        pltpu.VMEM((8, 8, TBLK + 1), jnp.float32),
        pltpu.SemaphoreType.DMA,
        pltpu.SemaphoreType.DMA,
        pltpu.SemaphoreType.DMA,
    ],
    compiler_params=pltpu.CompilerParams(
        use_tc_tiling_on_sc=False, needs_layout_passes=False),
)
def _embed_gather(idxt_hbm, table_hbm, out_hbm, idxt_v, rows0_v, rows1_v,
                  t0_v, t1_v, sem0, sem1, osem):
    wid = lax.axis_index("s") * NC + lax.axis_index("c")

    pltpu.sync_copy(idxt_hbm.at[:, pl.ds(wid * TBLK, TBLK)], idxt_v)

    def issue(j, buf, sem):
        pltpu.async_copy(table_hbm.at[idxt_v.at[j]], buf, sem)

    def drain(buf, sem):
        pltpu.make_async_copy(table_hbm.at[idxt_v.at[0]], buf, sem).wait()

    iota = lax.iota(jnp.int32, 16)
    icl = jnp.bitwise_and(iota, 7)
    ic8_base = jnp.right_shift(iota, 3)
    igrp_c8 = [ic8_base + 2 * k for k in range(EMBED // 16)]
    igrp_cl = [icl for _ in range(EMBED // 16)]

    def transform(rows_v, t_v):
        def row_body(tt, _):
            tv = jnp.full((16,), 0, jnp.int32) + tt
            for k in range(EMBED // 16):
                v = rows_v[tt, pl.ds(k * 16, 16)]
                plsc.store_scatter(
                    t_v, [igrp_c8[k], igrp_cl[k], tv], v * SCALE)
            return 0
        lax.fori_loop(0, TBLK, row_body, 0, unroll=4)

    def store(j, t_v):
        pltpu.async_copy(t_v.at[:, :, pl.ds(0, TBLK)],
                         out_hbm.at[j, :, wid], osem)

    def drain_store(t_v):
        pltpu.make_async_copy(out_hbm.at[0, :, wid],
                              t_v.at[:, :, pl.ds(0, TBLK)], osem).wait()

    issue(0, rows0_v, sem0)

    def pair_body(t, _):
        j0 = 2 * t
        issue(j0 + 1, rows1_v, sem1)
        drain(rows0_v, sem0)

        @pl.when(t > 0)
        def _():
            drain_store(t0_v)

        transform(rows0_v, t0_v)
        store(j0, t0_v)

        @pl.when(j0 + 2 < COLS)
        def _():
            issue(j0 + 2, rows0_v, sem0)

        drain(rows1_v, sem1)

        @pl.when(t > 0)
        def _():
            drain_store(t1_v)

        transform(rows1_v, t1_v)
        store(j0 + 1, t1_v)
        return 0

    lax.fori_loop(0, COLS // 2, pair_body, 0)
    drain_store(t0_v)
    drain_store(t1_v)


def kernel(tokens, table):
    tp = _retile(table)
    out5 = _embed_gather(tokens.astype(jnp.int32).T, tp)
    return out5.transpose(2, 4, 0, 1, 3).reshape(ROWS, COLS, EMBED)
